# baseline (speedup 1.0000x reference)
import jax
import jax.numpy as jnp
from jax import lax
from jax.experimental import pallas as pl
from jax.experimental.pallas import tpu as pltpu

_BCOLS = 16768


def _linear_kernel(x_ref, w_ref, b_ref, o_ref):
    o_ref[...] = lax.dot_general(
        w_ref[...], x_ref[...],
        (((1,), (0,)), ((), ())),
        preferred_element_type=jnp.float32,
    ) + b_ref[...]


def kernel(x, edge_index, W, b):
    n, h = x.shape
    out_dim = W.shape[0]
    xt = x.T
    b2 = b.reshape(out_dim, 1)
    out_t = pl.pallas_call(
        _linear_kernel,
        grid=(pl.cdiv(n, _BCOLS),),
        in_specs=[
            pl.BlockSpec((h, _BCOLS), lambda i: (0, i)),
            pl.BlockSpec((out_dim, h), lambda i: (0, 0)),
            pl.BlockSpec((out_dim, 1), lambda i: (0, 0)),
        ],
        out_specs=pl.BlockSpec((out_dim, _BCOLS), lambda i: (0, i)),
        out_shape=jax.ShapeDtypeStruct((out_dim, n), jnp.float32),
        compiler_params=pltpu.CompilerParams(
            dimension_semantics=("parallel",),
        ),
    )(xt, W, b2)
    return out_t.T


# speedup vs baseline: 1.1718x; 1.1718x over previous
import jax
import jax.numpy as jnp
from jax import lax
from jax.experimental import pallas as pl
from jax.experimental.pallas import tpu as pltpu

_BCOLS = 25088


def _linear_kernel(x_ref, w_ref, b_ref, o_ref):
    o_ref[...] = lax.dot_general(
        w_ref[...], x_ref[...],
        (((1,), (0,)), ((), ())),
        preferred_element_type=jnp.float32,
    ) + b_ref[...]


def kernel(x, edge_index, W, b):
    n, h = x.shape
    out_dim = W.shape[0]
    xt = x.T
    b2 = b.reshape(out_dim, 1)
    out_t = pl.pallas_call(
        _linear_kernel,
        grid=(pl.cdiv(n, _BCOLS),),
        in_specs=[
            pl.BlockSpec((h, _BCOLS), lambda i: (0, i)),
            pl.BlockSpec((out_dim, h), lambda i: (0, 0)),
            pl.BlockSpec((out_dim, 1), lambda i: (0, 0)),
        ],
        out_specs=pl.BlockSpec((out_dim, _BCOLS), lambda i: (0, i)),
        out_shape=jax.ShapeDtypeStruct((out_dim, n), jnp.float32),
        compiler_params=pltpu.CompilerParams(
            dimension_semantics=("parallel",),
        ),
    )(xt, W, b2)
    return out_t.T
